# Initial kernel scaffold; baseline (speedup 1.0000x reference)
import jax
import jax.numpy as jnp
from jax.experimental import pallas as pl


def kernel(feat, label):
    raise NotImplementedError("write your pallas kernel here")


# speedup vs baseline: 1.3916x; 1.3916x over previous
import jax
import jax.numpy as jnp
from jax import lax
from jax.experimental import pallas as pl
from jax.experimental.pallas import tpu as pltpu

_N_ROW = 16384
_N_CLS = 4096
_N_FEAT = 4096
_NUM_POS = 4

_RB = 512
_CB = 512


def _segsum_body(label_ref, feat_ref, sums_ref, counts_ref):
    i = pl.program_id(0)
    j = pl.program_id(1)
    labels = label_ref[0, 0, :]
    cls_ids = jax.lax.broadcasted_iota(jnp.int32, (_CB, _RB), 0) + i * _CB
    onehot = (cls_ids == labels[None, :]).astype(jnp.float32)
    part = jnp.dot(onehot, feat_ref[...],
                   preferred_element_type=jnp.float32)
    cpart = jnp.sum(onehot, axis=1, keepdims=True)

    @pl.when(j == 0)
    def _():
        sums_ref[...] = jnp.zeros_like(sums_ref)
        counts_ref[...] = jnp.zeros_like(counts_ref)

    sums_ref[...] += part
    counts_ref[...] += cpart


def _segsum(feat, label2d):
    return pl.pallas_call(
        _segsum_body,
        grid=(_N_CLS // _CB, _N_ROW // _RB),
        in_specs=[
            pl.BlockSpec((1, 1, _RB), lambda i, j: (j, 0, 0)),
            pl.BlockSpec((_RB, _N_FEAT), lambda i, j: (j, 0)),
        ],
        out_specs=[
            pl.BlockSpec((_CB, _N_FEAT), lambda i, j: (i, 0)),
            pl.BlockSpec((_CB, 1), lambda i, j: (i, 0)),
        ],
        out_shape=[
            jax.ShapeDtypeStruct((_N_CLS, _N_FEAT), jnp.float32),
            jax.ShapeDtypeStruct((_N_CLS, 1), jnp.float32),
        ],
    )(label2d, feat)


_B = 512


def _loss_body(sums_ref, counts_ref, out_ref):
    pid = pl.program_id(0)
    cnt = counts_ref[...]
    inv = 1.0 / jnp.maximum(cnt, 1.0)
    mean = sums_ref[...] * inv
    mx = jnp.max(mean, axis=1, keepdims=True)
    lse = jnp.log(jnp.sum(jnp.exp(mean - mx), axis=1, keepdims=True)) + mx
    rows = lax.broadcasted_iota(jnp.int32, mean.shape, 0) + pid * _B
    cols = lax.broadcasted_iota(jnp.int32, mean.shape, 1)
    diag = jnp.sum(jnp.where(rows == cols, mean, 0.0), axis=1, keepdims=True)
    contrib = jnp.sum(cnt * (lse - diag))
    scale = 1.0 / (_N_ROW * (_N_ROW / _NUM_POS))

    @pl.when(pid == 0)
    def _():
        out_ref[0, 0] = 0.0

    out_ref[0, 0] += contrib * scale


def _loss(sums, counts):
    return pl.pallas_call(
        _loss_body,
        grid=(_N_CLS // _B,),
        in_specs=[
            pl.BlockSpec((_B, _N_FEAT), lambda i: (i, 0)),
            pl.BlockSpec((_B, 1), lambda i: (i, 0)),
        ],
        out_specs=pl.BlockSpec((1, 1), lambda i: (0, 0),
                               memory_space=pltpu.SMEM),
        out_shape=jax.ShapeDtypeStruct((1, 1), jnp.float32),
    )(sums, counts)


@jax.jit
def kernel(feat, label):
    label2d = label.astype(jnp.int32).reshape(_N_ROW // _RB, 1, _RB)
    sums, counts = _segsum(feat, label2d)
    out = _loss(sums, counts)
    return out[0, 0]
